# Initial kernel scaffold; baseline (speedup 1.0000x reference)
import jax
import jax.numpy as jnp
from jax.experimental import pallas as pl


def kernel(queries, keys):
    raise NotImplementedError("write your pallas kernel here")


# speedup vs baseline: 3.0767x; 3.0767x over previous
import functools

import jax
import jax.numpy as jnp
from jax import lax
from jax.experimental import pallas as pl
from jax.experimental.pallas import tpu as pltpu
from jax.experimental.pallas import tpu_sc as plsc

D = 512
TOPK = 10
INV_TEMP = 1.0 / 0.07
NEG = -1e30
KB = 2048
CHUNK = 128
CPB = KB // CHUNK
BIG_I = 1 << 30


def _normsq_body(x_ref, o_ref):
    x = x_ref[...]
    sq = x * x
    t = ((sq[:, 0:128] + sq[:, 128:256]) + sq[:, 256:384]) + sq[:, 384:512]
    rows = x.shape[0]
    t3 = t.reshape(rows, 16, 8)
    acc = t3[:, 0, :]
    for m in range(1, 16):
        acc = acc + t3[:, m, :]
    e = [acc[:, i:i + 1] for i in range(8)]
    left = (e[0] + e[4]) + (e[2] + e[6])
    right = (e[1] + e[5]) + (e[3] + e[7])
    o_ref[...] = left + right


def _rownorm_sq(x, rows_blk):
    rows = x.shape[0]
    grid = -(-rows // rows_blk)
    return pl.pallas_call(
        _normsq_body,
        grid=(grid,),
        in_specs=[pl.BlockSpec((rows_blk, D), lambda i: (i, 0))],
        out_specs=pl.BlockSpec((rows_blk, 1), lambda i: (i, 0)),
        out_shape=jax.ShapeDtypeStruct((rows, 1), jnp.float32),
    )(x)


def _sim_body(nk, kvalid, q_ref, k_ref, s_ref, m_ref):
    ki = pl.program_id(1)
    q = q_ref[...]
    k = k_ref[...]
    s = lax.dot_general(q, k, (((1,), (1,)), ((), ())),
                        preferred_element_type=jnp.float32) * INV_TEMP
    qt = s.shape[0]

    def chunkmax(sv):
        return jnp.max(sv.reshape(qt, CPB, CHUNK), axis=2).reshape(1, qt, CPB)

    s_ref[...] = s
    m_ref[...] = chunkmax(s)

    @pl.when(ki == nk - 1)
    def _():
        col = lax.broadcasted_iota(jnp.int32, s.shape, 1) + (nk - 1) * KB
        s2 = jnp.where(col < kvalid, s, NEG)
        s_ref[...] = s2
        m_ref[...] = chunkmax(s2)


def _select_body(nchunk, m_ref, gi_ref):
    mm = m_ref[...]
    qb = mm.shape[0]
    ch = lax.broadcasted_iota(jnp.int32, mm.shape, 1)
    cols = []
    for _ in range(TOPK):
        mx = jnp.max(mm, axis=1, keepdims=True)
        sel = jnp.min(jnp.where(mm == mx, ch, BIG_I), axis=1, keepdims=True)
        cols.append(sel)
        mm = jnp.where(ch == sel, NEG, mm)
    cols.append(jnp.zeros((qb, 16 - TOPK), jnp.int32))
    cid = jnp.concatenate(cols, axis=1)
    row = pl.program_id(0) * qb + lax.broadcasted_iota(jnp.int32, (qb, 16), 0)
    gi_ref[...] = row * nchunk + cid


def _final_body(nchunk, kvalid, c_ref, g_ref, v_ref, i_ref):
    V = c_ref[...]
    gi = g_ref[...]
    qb = V.shape[0]
    rowbase = (pl.program_id(0) * qb
               + lax.broadcasted_iota(jnp.int32, (qb, 1), 0)) * nchunk
    lane = lax.broadcasted_iota(jnp.int32, (qb, CHUNK), 1)
    pieces = [(gi[:, j:j + 1] - rowbase) * CHUNK + lane for j in range(TOPK)]
    G = jnp.concatenate(pieces, axis=1)
    V = jnp.where(G < kvalid, V, NEG)
    vcols, icols = [], []
    for _ in range(TOPK):
        mx = jnp.max(V, axis=1, keepdims=True)
        gs = jnp.min(jnp.where(V == mx, G, BIG_I), axis=1, keepdims=True)
        vcols.append(mx)
        icols.append(gs)
        V = jnp.where(G == gs, NEG, V)
    v_ref[...] = jnp.concatenate(vcols, axis=1)
    i_ref[...] = jnp.concatenate(icols, axis=1)


def _sc_gather(table, g):
    B = g.shape[0]
    info = plsc.get_sparse_core_info()
    nw = info.num_cores * info.num_subcores
    bpw = B // nw
    SUB = 128
    nsub = bpw // SUB
    mesh = plsc.VectorSubcoreMesh(core_axis_name="c", subcore_axis_name="s")

    @functools.partial(
        pl.kernel,
        mesh=mesh,
        out_type=jax.ShapeDtypeStruct((B, CHUNK), jnp.float32),
        scratch_types=[
            pltpu.VMEM((SUB,), jnp.int32),
            pltpu.VMEM((SUB, CHUNK), jnp.float32),
            pltpu.SemaphoreType.DMA,
        ],
    )
    def k(table_hbm, g_hbm, out_hbm, idx_v, rows_v, sem):
        wid = lax.axis_index("s") * info.num_cores + lax.axis_index("c")
        base0 = wid * bpw
        for b in range(nsub):
            base = base0 + b * SUB
            pltpu.sync_copy(g_hbm.at[pl.ds(base, SUB)], idx_v)
            pltpu.async_copy(table_hbm.at[idx_v], rows_v, sem).wait()
            pltpu.sync_copy(rows_v, out_hbm.at[pl.ds(base, SUB)])

    return k(table, g)


def kernel(queries, keys):
    q_rows, d = queries.shape
    k_rows = keys.shape[0]
    assert d == D
    nk = -(-k_rows // KB)
    kp = nk * KB
    nchunk = kp // CHUNK

    qn = queries / (jnp.sqrt(_rownorm_sq(queries, 1024)) + 1e-8)
    kn = keys / (jnp.sqrt(_rownorm_sq(keys, KB)) + 1e-8)

    QT = 1024
    nq = q_rows // QT
    S, M3 = pl.pallas_call(
        functools.partial(_sim_body, nk, k_rows),
        grid=(nq, nk),
        in_specs=[pl.BlockSpec((QT, D), lambda qi, ki: (qi, 0)),
                  pl.BlockSpec((KB, D), lambda qi, ki: (ki, 0))],
        out_specs=[pl.BlockSpec((QT, KB), lambda qi, ki: (qi, ki)),
                   pl.BlockSpec((1, QT, CPB), lambda qi, ki: (ki, qi, 0))],
        out_shape=[jax.ShapeDtypeStruct((q_rows, kp), jnp.float32),
                   jax.ShapeDtypeStruct((nk, q_rows, CPB), jnp.float32)],
    )(qn, kn)

    M = M3.transpose(1, 0, 2).reshape(q_rows, nchunk)

    QB2 = 512
    gidx = pl.pallas_call(
        functools.partial(_select_body, nchunk),
        grid=(q_rows // QB2,),
        in_specs=[pl.BlockSpec((QB2, nchunk), lambda i: (i, 0))],
        out_specs=pl.BlockSpec((QB2, 16), lambda i: (i, 0)),
        out_shape=jax.ShapeDtypeStruct((q_rows, 16), jnp.int32),
    )(M)

    g = gidx[:, :TOPK].reshape(q_rows * TOPK)
    table = S.reshape(q_rows * nchunk, CHUNK)
    cand = _sc_gather(table, g).reshape(q_rows, TOPK * CHUNK)

    QB3 = 256
    vals, idx = pl.pallas_call(
        functools.partial(_final_body, nchunk, k_rows),
        grid=(q_rows // QB3,),
        in_specs=[pl.BlockSpec((QB3, TOPK * CHUNK), lambda i: (i, 0)),
                  pl.BlockSpec((QB3, 16), lambda i: (i, 0))],
        out_specs=[pl.BlockSpec((QB3, TOPK), lambda i: (i, 0)),
                   pl.BlockSpec((QB3, TOPK), lambda i: (i, 0))],
        out_shape=[jax.ShapeDtypeStruct((q_rows, TOPK), jnp.float32),
                   jax.ShapeDtypeStruct((q_rows, TOPK), jnp.int32)],
    )(cand, gidx)
    return vals, idx


# speedup vs baseline: 6.3909x; 2.0772x over previous
import functools

import jax
import jax.numpy as jnp
from jax import lax
from jax.experimental import pallas as pl
from jax.experimental.pallas import tpu as pltpu
from jax.experimental.pallas import tpu_sc as plsc

D = 512
TOPK = 10
INV_TEMP = 1.0 / 0.07
NEG = -1e30
KB = 2048
CHUNK = 128
CPB = KB // CHUNK
BIG_I = 1 << 30


def _normsq_body(x_ref, o_ref):
    x = x_ref[...]
    sq = x * x
    t = ((sq[:, 0:128] + sq[:, 128:256]) + sq[:, 256:384]) + sq[:, 384:512]
    rows = x.shape[0]
    tt = t.T.reshape(16, 8, rows)
    acc = tt[0]
    for m in range(1, 16):
        acc = acc + tt[m]
    e = [acc[i:i + 1, :] for i in range(8)]
    left = (e[0] + e[4]) + (e[2] + e[6])
    right = (e[1] + e[5]) + (e[3] + e[7])
    o_ref[...] = left + right


def _rownorm_sq(x, rows_blk):
    rows = x.shape[0]
    grid = -(-rows // rows_blk)
    out = pl.pallas_call(
        _normsq_body,
        grid=(grid,),
        in_specs=[pl.BlockSpec((rows_blk, D), lambda i: (i, 0))],
        out_specs=pl.BlockSpec((1, rows_blk), lambda i: (0, i)),
        out_shape=jax.ShapeDtypeStruct((1, grid * rows_blk), jnp.float32),
    )(x)
    return out.reshape(-1)[:rows]


def _sim_body(nk, kvalid, q_ref, k_ref, s_ref, m_ref):
    ki = pl.program_id(1)
    q = q_ref[...]
    k = k_ref[...]
    s = lax.dot_general(q, k, (((1,), (1,)), ((), ())),
                        preferred_element_type=jnp.float32) * INV_TEMP
    qt = s.shape[0]
    trows = qt // 8

    def store(sv):
        cols = []
        for c in range(CPB):
            blk = sv[:, c * CHUNK:(c + 1) * CHUNK]
            s_ref[:, c, :, :] = blk.reshape(trows, 8, CHUNK)
            cols.append(jnp.max(blk, axis=1, keepdims=True))
        m_ref[...] = jnp.concatenate(cols, axis=1).reshape(1, qt, CPB)

    @pl.when(ki < nk - 1)
    def _():
        store(s)

    @pl.when(ki == nk - 1)
    def _():
        col = lax.broadcasted_iota(jnp.int32, s.shape, 1) + (nk - 1) * KB
        store(jnp.where(col < kvalid, s, NEG))


def _select_body(nchunk, m_ref, gi_ref, ci_ref):
    mm = m_ref[...]
    qb = mm.shape[0]
    ch = lax.broadcasted_iota(jnp.int32, mm.shape, 1)
    cols = []
    for _ in range(TOPK):
        mx = jnp.max(mm, axis=1, keepdims=True)
        sel = jnp.min(jnp.where(mm == mx, ch, BIG_I), axis=1, keepdims=True)
        cols.append(sel)
        mm = jnp.where(ch == sel, NEG, mm)
    cols.append(jnp.zeros((qb, 16 - TOPK), jnp.int32))
    cid = jnp.concatenate(cols, axis=1)
    row = pl.program_id(0) * qb + lax.broadcasted_iota(jnp.int32, (qb, 16), 0)
    gi_ref[...] = ((row >> 3) * nchunk + cid) * 8 + (row & 7)
    ci_ref[...] = cid


def _final_body(kvalid, c_ref, ci_ref, v_ref, i_ref):
    V = c_ref[...]
    cid = ci_ref[...]
    qb = V.shape[0]
    lane = lax.broadcasted_iota(jnp.int32, (qb, CHUNK), 1)
    pieces = [cid[:, j:j + 1] * CHUNK + lane for j in range(TOPK)]
    G = jnp.concatenate(pieces, axis=1)
    V = jnp.where(G < kvalid, V, NEG)
    vcols, icols = [], []
    for _ in range(TOPK):
        mx = jnp.max(V, axis=1, keepdims=True)
        gs = jnp.min(jnp.where(V == mx, G, BIG_I), axis=1, keepdims=True)
        vcols.append(mx)
        icols.append(gs)
        V = jnp.where(G == gs, NEG, V)
    v_ref[...] = jnp.concatenate(vcols, axis=1)
    i_ref[...] = jnp.concatenate(icols, axis=1)


def _sc_gather(table, g):
    B = g.shape[0]
    info = plsc.get_sparse_core_info()
    nw = info.num_cores * info.num_subcores
    bpw = B // nw
    SUB = 128
    nsub = bpw // SUB
    mesh = plsc.VectorSubcoreMesh(core_axis_name="c", subcore_axis_name="s")

    @functools.partial(
        pl.kernel,
        mesh=mesh,
        out_type=jax.ShapeDtypeStruct((B, CHUNK), jnp.float32),
        scratch_types=[
            pltpu.VMEM((SUB,), jnp.int32),
            pltpu.VMEM((SUB, CHUNK), jnp.float32),
            pltpu.SemaphoreType.DMA,
        ],
    )
    def k(table_hbm, g_hbm, out_hbm, idx_v, rows_v, sem):
        wid = lax.axis_index("s") * info.num_cores + lax.axis_index("c")
        base0 = wid * bpw
        for b in range(nsub):
            base = base0 + b * SUB
            pltpu.sync_copy(g_hbm.at[pl.ds(base, SUB)], idx_v)
            pltpu.async_copy(table_hbm.at[idx_v], rows_v, sem).wait()
            pltpu.sync_copy(rows_v, out_hbm.at[pl.ds(base, SUB)])

    return k(table, g)


def kernel(queries, keys):
    q_rows, d = queries.shape
    k_rows = keys.shape[0]
    assert d == D
    nk = -(-k_rows // KB)
    kp = nk * KB
    nchunk = kp // CHUNK

    qn = queries / (jnp.sqrt(_rownorm_sq(queries, 1024))[:, None] + 1e-8)
    kn = keys / (jnp.sqrt(_rownorm_sq(keys, KB))[:, None] + 1e-8)

    QT = 1024
    nq = q_rows // QT
    S4, M3 = pl.pallas_call(
        functools.partial(_sim_body, nk, k_rows),
        grid=(nq, nk),
        in_specs=[pl.BlockSpec((QT, D), lambda qi, ki: (qi, 0)),
                  pl.BlockSpec((KB, D), lambda qi, ki: (ki, 0))],
        out_specs=[pl.BlockSpec((QT // 8, CPB, 8, CHUNK),
                                lambda qi, ki: (qi, ki, 0, 0)),
                   pl.BlockSpec((1, QT, CPB), lambda qi, ki: (ki, qi, 0))],
        out_shape=[jax.ShapeDtypeStruct((q_rows // 8, nchunk, 8, CHUNK),
                                        jnp.float32),
                   jax.ShapeDtypeStruct((nk, q_rows, CPB), jnp.float32)],
    )(qn, kn)

    M = M3.transpose(1, 0, 2).reshape(q_rows, nchunk)

    QB2 = 512
    gidx, cid = pl.pallas_call(
        functools.partial(_select_body, nchunk),
        grid=(q_rows // QB2,),
        in_specs=[pl.BlockSpec((QB2, nchunk), lambda i: (i, 0))],
        out_specs=[pl.BlockSpec((QB2, 16), lambda i: (i, 0)),
                   pl.BlockSpec((QB2, 16), lambda i: (i, 0))],
        out_shape=[jax.ShapeDtypeStruct((q_rows, 16), jnp.int32),
                   jax.ShapeDtypeStruct((q_rows, 16), jnp.int32)],
    )(M)

    g = gidx[:, :TOPK].reshape(q_rows * TOPK)
    table = S4.reshape(q_rows * nchunk, CHUNK)
    cand = _sc_gather(table, g).reshape(q_rows, TOPK * CHUNK)

    QB3 = 256
    vals, idx = pl.pallas_call(
        functools.partial(_final_body, k_rows),
        grid=(q_rows // QB3,),
        in_specs=[pl.BlockSpec((QB3, TOPK * CHUNK), lambda i: (i, 0)),
                  pl.BlockSpec((QB3, 16), lambda i: (i, 0))],
        out_specs=[pl.BlockSpec((QB3, TOPK), lambda i: (i, 0)),
                   pl.BlockSpec((QB3, TOPK), lambda i: (i, 0))],
        out_shape=[jax.ShapeDtypeStruct((q_rows, TOPK), jnp.float32),
                   jax.ShapeDtypeStruct((q_rows, TOPK), jnp.int32)],
    )(cand, cid)
    return vals, idx


# speedup vs baseline: 6.6406x; 1.0391x over previous
import functools

import jax
import jax.numpy as jnp
from jax import lax
from jax.experimental import pallas as pl
from jax.experimental.pallas import tpu as pltpu
from jax.experimental.pallas import tpu_sc as plsc

D = 512
TOPK = 10
INV_TEMP = 1.0 / 0.07
NEG = -1e30
KB = 1024
CHUNK = 128
CPB = KB // CHUNK
BIG_I = 1 << 30


def _normsq_body(x_ref, o_ref):
    x = x_ref[...]
    sq = x * x
    t = ((sq[:, 0:128] + sq[:, 128:256]) + sq[:, 256:384]) + sq[:, 384:512]
    rows = x.shape[0]
    tt = t.T.reshape(16, 8, rows)
    acc = tt[0]
    for m in range(1, 16):
        acc = acc + tt[m]
    e = [acc[i:i + 1, :] for i in range(8)]
    left = (e[0] + e[4]) + (e[2] + e[6])
    right = (e[1] + e[5]) + (e[3] + e[7])
    o_ref[...] = left + right


def _rownorm_sq(x, rows_blk):
    rows = x.shape[0]
    grid = -(-rows // rows_blk)
    out = pl.pallas_call(
        _normsq_body,
        grid=(grid,),
        in_specs=[pl.BlockSpec((rows_blk, D), lambda i: (i, 0))],
        out_specs=pl.BlockSpec((1, rows_blk), lambda i: (0, i)),
        out_shape=jax.ShapeDtypeStruct((1, grid * rows_blk), jnp.float32),
    )(x)
    return out.reshape(-1)[:rows]


def _sim_body(nk, kvalid, q_ref, k_ref, d_ref, s_ref, m_ref):
    ki = pl.program_id(1)
    q = q_ref[...]
    k = k_ref[...] / d_ref[...]
    s = lax.dot_general(q, k, (((1,), (1,)), ((), ())),
                        preferred_element_type=jnp.float32) * INV_TEMP
    qt = s.shape[0]
    trows = qt // 8

    def store(sv):
        cols = []
        for c in range(CPB):
            blk = sv[:, c * CHUNK:(c + 1) * CHUNK]
            s_ref[:, c, :, :] = blk.reshape(trows, 8, CHUNK)
            cols.append(jnp.max(blk, axis=1, keepdims=True))
        m_ref[...] = jnp.concatenate(cols, axis=1).reshape(1, qt, CPB)

    @pl.when(ki < nk - 1)
    def _():
        store(s)

    @pl.when(ki == nk - 1)
    def _():
        col = lax.broadcasted_iota(jnp.int32, s.shape, 1) + (nk - 1) * KB
        store(jnp.where(col < kvalid, s, NEG))


def _select_body(nchunk, m_ref, gi_ref, ci_ref):
    mm = m_ref[...]
    qb = mm.shape[0]
    ch = lax.broadcasted_iota(jnp.int32, mm.shape, 1)
    cols = []
    for _ in range(TOPK):
        mx = jnp.max(mm, axis=1, keepdims=True)
        sel = jnp.min(jnp.where(mm == mx, ch, BIG_I), axis=1, keepdims=True)
        cols.append(sel)
        mm = jnp.where(ch == sel, NEG, mm)
    cols.append(jnp.zeros((qb, 16 - TOPK), jnp.int32))
    cid = jnp.concatenate(cols, axis=1)
    row = pl.program_id(0) * qb + lax.broadcasted_iota(jnp.int32, (qb, 16), 0)
    gi_ref[...] = ((row >> 3) * nchunk + cid) * 8 + (row & 7)
    ci_ref[...] = cid


def _final_body(kvalid, c_ref, ci_ref, v_ref, i_ref):
    V = c_ref[...]
    cid = ci_ref[...]
    qb = V.shape[0]
    lane = lax.broadcasted_iota(jnp.int32, (qb, CHUNK), 1)
    pieces = [cid[:, j:j + 1] * CHUNK + lane for j in range(TOPK)]
    G = jnp.concatenate(pieces, axis=1)
    V = jnp.where(G < kvalid, V, NEG)
    vcols, icols = [], []
    for _ in range(TOPK):
        mx = jnp.max(V, axis=1, keepdims=True)
        gs = jnp.min(jnp.where(V == mx, G, BIG_I), axis=1, keepdims=True)
        vcols.append(mx)
        icols.append(gs)
        V = jnp.where(G == gs, NEG, V)
    v_ref[...] = jnp.concatenate(vcols, axis=1)
    i_ref[...] = jnp.concatenate(icols, axis=1)


def _sc_gather(table, g):
    B = g.shape[0]
    info = plsc.get_sparse_core_info()
    nw = info.num_cores * info.num_subcores
    bpw = B // nw
    SUB = 128
    nsub = bpw // SUB
    mesh = plsc.VectorSubcoreMesh(core_axis_name="c", subcore_axis_name="s")

    @functools.partial(
        pl.kernel,
        mesh=mesh,
        out_type=jax.ShapeDtypeStruct((B, CHUNK), jnp.float32),
        scratch_types=[
            pltpu.VMEM((SUB,), jnp.int32),
            pltpu.VMEM((SUB, CHUNK), jnp.float32),
            pltpu.SemaphoreType.DMA,
        ],
    )
    def k(table_hbm, g_hbm, out_hbm, idx_v, rows_v, sem):
        wid = lax.axis_index("s") * info.num_cores + lax.axis_index("c")
        base0 = wid * bpw
        for b in range(nsub):
            base = base0 + b * SUB
            pltpu.sync_copy(g_hbm.at[pl.ds(base, SUB)], idx_v)
            pltpu.async_copy(table_hbm.at[idx_v], rows_v, sem).wait()
            pltpu.sync_copy(rows_v, out_hbm.at[pl.ds(base, SUB)])

    return k(table, g)


def kernel(queries, keys):
    q_rows, d = queries.shape
    k_rows = keys.shape[0]
    assert d == D
    nk = -(-k_rows // KB)
    kp = nk * KB
    nchunk = kp // CHUNK

    qn = queries / (jnp.sqrt(_rownorm_sq(queries, 1024))[:, None] + 1e-8)
    kd = (jnp.sqrt(_rownorm_sq(keys, KB)) + 1e-8)[:, None]

    QT = 2048
    nq = q_rows // QT
    S4, M3 = pl.pallas_call(
        functools.partial(_sim_body, nk, k_rows),
        grid=(nq, nk),
        in_specs=[pl.BlockSpec((QT, D), lambda qi, ki: (qi, 0)),
                  pl.BlockSpec((KB, D), lambda qi, ki: (ki, 0)),
                  pl.BlockSpec((KB, 1), lambda qi, ki: (ki, 0))],
        out_specs=[pl.BlockSpec((QT // 8, CPB, 8, CHUNK),
                                lambda qi, ki: (qi, ki, 0, 0)),
                   pl.BlockSpec((1, QT, CPB), lambda qi, ki: (ki, qi, 0))],
        out_shape=[jax.ShapeDtypeStruct((q_rows // 8, nchunk, 8, CHUNK),
                                        jnp.float32),
                   jax.ShapeDtypeStruct((nk, q_rows, CPB), jnp.float32)],
    )(qn, keys, kd)

    M = M3.transpose(1, 0, 2).reshape(q_rows, nchunk)

    QB2 = 512
    gidx, cid = pl.pallas_call(
        functools.partial(_select_body, nchunk),
        grid=(q_rows // QB2,),
        in_specs=[pl.BlockSpec((QB2, nchunk), lambda i: (i, 0))],
        out_specs=[pl.BlockSpec((QB2, 16), lambda i: (i, 0)),
                   pl.BlockSpec((QB2, 16), lambda i: (i, 0))],
        out_shape=[jax.ShapeDtypeStruct((q_rows, 16), jnp.int32),
                   jax.ShapeDtypeStruct((q_rows, 16), jnp.int32)],
    )(M)

    g = gidx[:, :TOPK].reshape(q_rows * TOPK)
    table = S4.reshape(q_rows * nchunk, CHUNK)
    cand = _sc_gather(table, g).reshape(q_rows, TOPK * CHUNK)

    QB3 = 256
    vals, idx = pl.pallas_call(
        functools.partial(_final_body, k_rows),
        grid=(q_rows // QB3,),
        in_specs=[pl.BlockSpec((QB3, TOPK * CHUNK), lambda i: (i, 0)),
                  pl.BlockSpec((QB3, 16), lambda i: (i, 0))],
        out_specs=[pl.BlockSpec((QB3, TOPK), lambda i: (i, 0)),
                   pl.BlockSpec((QB3, TOPK), lambda i: (i, 0))],
        out_shape=[jax.ShapeDtypeStruct((q_rows, TOPK), jnp.float32),
                   jax.ShapeDtypeStruct((q_rows, TOPK), jnp.int32)],
    )(cand, cid)
    return vals, idx


# speedup vs baseline: 8.9522x; 1.3481x over previous
import functools

import jax
import jax.numpy as jnp
from jax import lax
from jax.experimental import pallas as pl
from jax.experimental.pallas import tpu as pltpu
from jax.experimental.pallas import tpu_sc as plsc

D = 512
TOPK = 10
INV_TEMP = 1.0 / 0.07
NEG = -1e30
KB = 1024
CHUNK = 128
CPB = KB // CHUNK
BIG_I = 1 << 30


def _normsq_body(x_ref, o_ref):
    x = x_ref[...]
    sq = x * x
    t = ((sq[:, 0:128] + sq[:, 128:256]) + sq[:, 256:384]) + sq[:, 384:512]
    rows = x.shape[0]
    tt = t.T.reshape(16, 8, rows)
    acc = tt[0]
    for m in range(1, 16):
        acc = acc + tt[m]
    e = [acc[i:i + 1, :] for i in range(8)]
    left = (e[0] + e[4]) + (e[2] + e[6])
    right = (e[1] + e[5]) + (e[3] + e[7])
    o_ref[...] = left + right


def _rownorm_sq(x, rows_blk):
    rows = x.shape[0]
    grid = -(-rows // rows_blk)
    out = pl.pallas_call(
        _normsq_body,
        grid=(grid,),
        in_specs=[pl.BlockSpec((rows_blk, D), lambda i: (i, 0))],
        out_specs=pl.BlockSpec((1, rows_blk), lambda i: (0, i)),
        out_shape=jax.ShapeDtypeStruct((1, grid * rows_blk), jnp.float32),
    )(x)
    return out.reshape(-1)[:rows]


def _sim_body(nk, kvalid, q_ref, k_ref, d_ref, s_ref, m_ref):
    ki = pl.program_id(1)
    q = q_ref[...]
    k = k_ref[...] / d_ref[...]
    qt = q.shape[0]
    trows = qt // 8

    DOTW = 512
    CPD = DOTW // CHUNK

    def store(masked):
        cols = []
        for g in range(KB // DOTW):
            kc = k[g * DOTW:(g + 1) * DOTW, :]
            sg = lax.dot_general(q, kc, (((1,), (1,)), ((), ())),
                                 preferred_element_type=jnp.float32)
            sg = sg * INV_TEMP
            if masked:
                col = (lax.broadcasted_iota(jnp.int32, sg.shape, 1)
                       + (nk - 1) * KB + g * DOTW)
                sg = jnp.where(col < kvalid, sg, NEG)
            for cc in range(CPD):
                c = g * CPD + cc
                blk = sg[:, cc * CHUNK:(cc + 1) * CHUNK]
                s_ref[:, c, :, :] = blk.reshape(trows, 8, CHUNK)
                cols.append(jnp.max(blk, axis=1, keepdims=True))
        m_ref[...] = jnp.concatenate(cols, axis=1).reshape(1, qt, CPB)

    @pl.when(ki < nk - 1)
    def _():
        store(False)

    @pl.when(ki == nk - 1)
    def _():
        store(True)


def _select_body(nchunk, m_ref, gi_ref, ci_ref):
    mm = m_ref[...]
    qb = mm.shape[0]
    ch = lax.broadcasted_iota(jnp.int32, mm.shape, 1)
    cols = []
    for _ in range(TOPK):
        mx = jnp.max(mm, axis=1, keepdims=True)
        sel = jnp.min(jnp.where(mm == mx, ch, BIG_I), axis=1, keepdims=True)
        cols.append(sel)
        mm = jnp.where(ch == sel, NEG, mm)
    cols.append(jnp.zeros((qb, 16 - TOPK), jnp.int32))
    cid = jnp.concatenate(cols, axis=1)
    row = pl.program_id(0) * qb + lax.broadcasted_iota(jnp.int32, (qb, 16), 0)
    gi_ref[...] = ((row >> 3) * nchunk + cid) * 8 + (row & 7)
    ci_ref[...] = cid


def _final_body(kvalid, c_ref, ci_ref, v_ref, i_ref):
    V = c_ref[...]
    cid = ci_ref[...]
    qb = V.shape[0]
    lane = lax.broadcasted_iota(jnp.int32, (qb, CHUNK), 1)
    pieces = [cid[:, j:j + 1] * CHUNK + lane for j in range(TOPK)]
    G = jnp.concatenate(pieces, axis=1)
    V = jnp.where(G < kvalid, V, NEG)
    vcols, icols = [], []
    for _ in range(TOPK):
        mx = jnp.max(V, axis=1, keepdims=True)
        gs = jnp.min(jnp.where(V == mx, G, BIG_I), axis=1, keepdims=True)
        vcols.append(mx)
        icols.append(gs)
        V = jnp.where(G == gs, NEG, V)
    v_ref[...] = jnp.concatenate(vcols, axis=1)
    i_ref[...] = jnp.concatenate(icols, axis=1)


def _sc_gather(table, g):
    B = g.shape[0]
    info = plsc.get_sparse_core_info()
    nw = info.num_cores * info.num_subcores
    bpw = B // nw
    SUB = 128
    nsub = bpw // SUB
    mesh = plsc.VectorSubcoreMesh(core_axis_name="c", subcore_axis_name="s")

    @functools.partial(
        pl.kernel,
        mesh=mesh,
        out_type=jax.ShapeDtypeStruct((B, CHUNK), jnp.float32),
        scratch_types=[
            pltpu.VMEM((SUB,), jnp.int32),
            pltpu.VMEM((SUB, CHUNK), jnp.float32),
            pltpu.SemaphoreType.DMA,
        ],
    )
    def k(table_hbm, g_hbm, out_hbm, idx_v, rows_v, sem):
        wid = lax.axis_index("s") * info.num_cores + lax.axis_index("c")
        base0 = wid * bpw
        for b in range(nsub):
            base = base0 + b * SUB
            pltpu.sync_copy(g_hbm.at[pl.ds(base, SUB)], idx_v)
            pltpu.async_copy(table_hbm.at[idx_v], rows_v, sem).wait()
            pltpu.sync_copy(rows_v, out_hbm.at[pl.ds(base, SUB)])

    return k(table, g)


def kernel(queries, keys):
    q_rows, d = queries.shape
    k_rows = keys.shape[0]
    assert d == D
    nk = -(-k_rows // KB)
    kp = nk * KB
    nchunk = kp // CHUNK

    qn = queries / (jnp.sqrt(_rownorm_sq(queries, 1024))[:, None] + 1e-8)
    kd = (jnp.sqrt(_rownorm_sq(keys, KB)) + 1e-8)[:, None]

    QT = 2048
    nq = q_rows // QT
    S4, M3 = pl.pallas_call(
        functools.partial(_sim_body, nk, k_rows),
        grid=(nq, nk),
        in_specs=[pl.BlockSpec((QT, D), lambda qi, ki: (qi, 0)),
                  pl.BlockSpec((KB, D), lambda qi, ki: (ki, 0)),
                  pl.BlockSpec((KB, 1), lambda qi, ki: (ki, 0))],
        out_specs=[pl.BlockSpec((QT // 8, CPB, 8, CHUNK),
                                lambda qi, ki: (qi, ki, 0, 0)),
                   pl.BlockSpec((1, QT, CPB), lambda qi, ki: (ki, qi, 0))],
        out_shape=[jax.ShapeDtypeStruct((q_rows // 8, nchunk, 8, CHUNK),
                                        jnp.float32),
                   jax.ShapeDtypeStruct((nk, q_rows, CPB), jnp.float32)],
    )(qn, keys, kd)

    M = M3.transpose(1, 0, 2).reshape(q_rows, nchunk)

    QB2 = 512
    gidx, cid = pl.pallas_call(
        functools.partial(_select_body, nchunk),
        grid=(q_rows // QB2,),
        in_specs=[pl.BlockSpec((QB2, nchunk), lambda i: (i, 0))],
        out_specs=[pl.BlockSpec((QB2, 16), lambda i: (i, 0)),
                   pl.BlockSpec((QB2, 16), lambda i: (i, 0))],
        out_shape=[jax.ShapeDtypeStruct((q_rows, 16), jnp.int32),
                   jax.ShapeDtypeStruct((q_rows, 16), jnp.int32)],
    )(M)

    g = gidx[:, :TOPK].reshape(q_rows * TOPK)
    table = S4.reshape(q_rows * nchunk, CHUNK)
    cand = _sc_gather(table, g).reshape(q_rows, TOPK * CHUNK)

    QB3 = 256
    vals, idx = pl.pallas_call(
        functools.partial(_final_body, k_rows),
        grid=(q_rows // QB3,),
        in_specs=[pl.BlockSpec((QB3, TOPK * CHUNK), lambda i: (i, 0)),
                  pl.BlockSpec((QB3, 16), lambda i: (i, 0))],
        out_specs=[pl.BlockSpec((QB3, TOPK), lambda i: (i, 0)),
                   pl.BlockSpec((QB3, TOPK), lambda i: (i, 0))],
        out_shape=[jax.ShapeDtypeStruct((q_rows, TOPK), jnp.float32),
                   jax.ShapeDtypeStruct((q_rows, TOPK), jnp.int32)],
    )(cand, cid)
    return vals, idx
